# baseline (speedup 1.0000x reference)
import functools

import jax
import jax.numpy as jnp
from jax import lax
from jax.experimental import pallas as pl
from jax.experimental.pallas import tpu as pltpu
from jax.experimental.pallas import tpu_sc as plsc

N_CLASSES = 5
D = 128
NU = 5000
NI = 5000
E = 320000
ED = 100000

NCORES = 2
NSUB = 16
L = 16

E_PER_TILE = E // (NCORES * NSUB)
CHUNK = 80
NCHUNKS = E_PER_TILE // CHUNK

FZ = 320

DTILE = 3136
DCHUNK = 112
DNCH = DTILE // DCHUNK

_MESH = plsc.VectorSubcoreMesh(
    core_axis_name="c", subcore_axis_name="s",
    num_cores=NCORES, num_subcores=NSUB)

_f32 = jnp.float32
_i32 = jnp.int32


def _dg(v, idx):
    return lax.gather(
        v, idx.reshape(L, 1),
        lax.GatherDimensionNumbers(
            offset_dims=(), collapsed_slice_dims=(0,), start_index_map=(0,)),
        slice_sizes=(1,),
        mode=lax.GatherScatterMode.PROMISE_IN_BOUNDS)


@functools.partial(
    pl.kernel,
    out_type=(
        jax.ShapeDtypeStruct((NCORES, NU, D), _f32),
        jax.ShapeDtypeStruct((NCORES, NI, D), _f32),
    ),
    mesh=_MESH,
    scratch_types=[
        pltpu.VMEM_SHARED((NU, D), _f32),
        pltpu.VMEM_SHARED((NI, D), _f32),
        pltpu.VMEM((4, CHUNK), _i32),
        pltpu.VMEM((4, CHUNK), _i32),
        pltpu.VMEM((4, CHUNK), _i32),
        pltpu.VMEM((2, CHUNK, D), _f32),
        pltpu.SemaphoreType.DMA((4,)),
        pltpu.SemaphoreType.DMA((2,)),
    ],
)
def _deg_kernel(eu, ei, et, zc, dpu, dpi, tabu, tabi, ubufs, ibufs, tbufs,
                msgs, semL, semS):
    c = lax.axis_index("c")
    s = lax.axis_index("s")
    zb = jnp.minimum(s * FZ, NU - FZ)
    pltpu.sync_copy(zc.at[pl.ds(zb, FZ)], tabu.at[pl.ds(zb, FZ)])
    pltpu.sync_copy(zc.at[pl.ds(zb, FZ)], tabi.at[pl.ds(zb, FZ)])
    pltpu.sync_copy(zc.at[pl.ds(0, CHUNK)], msgs.at[0])
    pltpu.sync_copy(zc.at[pl.ds(0, CHUNK)], msgs.at[1])
    plsc.subcore_barrier()

    tile_base = (c * NSUB + s) * E_PER_TILE
    iota = lax.iota(_i32, L)

    def L_cp(k):
        sl = lax.rem(k, 4)
        base = tile_base + k * CHUNK
        return (
            pltpu.make_async_copy(et.at[pl.ds(base, CHUNK)], tbufs.at[sl],
                                  semL.at[sl]),
            pltpu.make_async_copy(eu.at[pl.ds(base, CHUNK)], ubufs.at[sl],
                                  semL.at[sl]),
            pltpu.make_async_copy(ei.at[pl.ds(base, CHUNK)], ibufs.at[sl],
                                  semL.at[sl]),
        )

    def issue_L(k):
        for cp in L_cp(k):
            cp.start()

    def wait_L(k):
        for cp in L_cp(k):
            cp.wait()

    def issue_S(k):
        sl = lax.rem(k, 4)
        b = lax.rem(k, 2)
        pltpu.async_copy(msgs.at[b], tabu.at[ubufs.at[sl]], semS.at[b],
                         add=True)
        pltpu.async_copy(msgs.at[b], tabi.at[ibufs.at[sl]], semS.at[b],
                         add=True)

    def wait_S(k):
        sl = lax.rem(k, 4)
        b = lax.rem(k, 2)
        pltpu.make_async_copy(msgs.at[b], tabu.at[ubufs.at[sl]],
                              semS.at[b]).wait()
        pltpu.make_async_copy(msgs.at[b], tabi.at[ibufs.at[sl]],
                              semS.at[b]).wait()

    issue_L(0)
    issue_L(1)

    def body(k, carry):
        sl = lax.rem(k, 4)
        b = lax.rem(k, 2)
        wait_L(k)

        @pl.when(k >= 2)
        def _():
            wait_S(k - 2)

        for g in range(CHUNK // L):
            et16 = tbufs[sl, pl.ds(g * L, L)]
            for l in range(L):
                et_sp = _dg(et16, jnp.full((L,), l, _i32))
                msgs[b, g * L + l, pl.ds(0, L)] = jnp.where(
                    iota == et_sp, 1.0, 0.0)
        issue_S(k)

        @pl.when(k + 2 < NCHUNKS)
        def _():
            issue_L(k + 2)

        return carry

    lax.fori_loop(0, NCHUNKS, body, 0)
    wait_S(NCHUNKS - 2)
    wait_S(NCHUNKS - 1)
    plsc.subcore_barrier()
    pltpu.sync_copy(tabu.at[pl.ds(zb, FZ)], dpu.at[c, pl.ds(zb, FZ)])
    pltpu.sync_copy(tabi.at[pl.ds(zb, FZ)], dpi.at[c, pl.ds(zb, FZ)])


@functools.partial(
    pl.kernel,
    out_type=jax.ShapeDtypeStruct((E, L), _f32),
    mesh=_MESH,
    scratch_types=[
        pltpu.VMEM((4, CHUNK), _i32),
        pltpu.VMEM((4, CHUNK), _i32),
        pltpu.VMEM((4, CHUNK), _i32),
        pltpu.VMEM((4, CHUNK), _i32),
        pltpu.VMEM((2, CHUNK, D), _f32),
        pltpu.VMEM((2, CHUNK, D), _f32),
        pltpu.VMEM((2, CHUNK, L), _f32),
        pltpu.SemaphoreType.DMA((4,)),
        pltpu.SemaphoreType.DMA((2,)),
        pltpu.SemaphoreType.DMA((2,)),
    ],
)
def _norm_kernel(eu, ei, et, degux, degix, nrm, ebufs, tbufs, fubufs, fibufs,
                 durows, dirows, nbufs, semL, semG, semS):
    c = lax.axis_index("c")
    s = lax.axis_index("s")
    tile_base = (c * NSUB + s) * E_PER_TILE

    def L_cp(k):
        sl = lax.rem(k, 4)
        base = tile_base + k * CHUNK
        return (
            pltpu.make_async_copy(et.at[pl.ds(base, CHUNK)], tbufs.at[sl],
                                  semL.at[sl]),
            pltpu.make_async_copy(eu.at[pl.ds(base, CHUNK)], ebufs.at[sl],
                                  semL.at[sl]),
        )

    def Li_cp(k):
        sl = lax.rem(k, 4)
        base = tile_base + k * CHUNK
        return pltpu.make_async_copy(ei.at[pl.ds(base, CHUNK)],
                                     ebufs.at[sl], semL.at[sl])

    def compute_f(k):
        sl = lax.rem(k, 4)
        for g in range(CHUNK // L):
            t16 = tbufs[sl, pl.ds(g * L, L)]
            u16 = ebufs[sl, pl.ds(g * L, L)]
            fubufs[sl, pl.ds(g * L, L)] = t16 * NU + u16
        Li_cp(k).start()
        Li_cp(k).wait()
        for g in range(CHUNK // L):
            t16 = tbufs[sl, pl.ds(g * L, L)]
            i16 = ebufs[sl, pl.ds(g * L, L)]
            fibufs[sl, pl.ds(g * L, L)] = t16 * NU + i16

    def G_cp(k):
        sl = lax.rem(k, 4)
        b = lax.rem(k, 2)
        return (
            pltpu.make_async_copy(degux.at[fubufs.at[sl]], durows.at[b],
                                  semG.at[b]),
            pltpu.make_async_copy(degix.at[fibufs.at[sl]], dirows.at[b],
                                  semG.at[b]),
        )

    def S_cp(k):
        b = lax.rem(k, 2)
        base = tile_base + k * CHUNK
        return pltpu.make_async_copy(nbufs.at[b],
                                     nrm.at[pl.ds(base, CHUNK)], semS.at[b])

    for cp in L_cp(0):
        cp.start()
    for cp in L_cp(0):
        cp.wait()
    compute_f(0)
    for cp in G_cp(0):
        cp.start()
    for cp in L_cp(1):
        cp.start()

    def body(k, carry):
        b = lax.rem(k, 2)
        for cp in G_cp(k):
            cp.wait()

        @pl.when(k + 1 < NCHUNKS)
        def _():
            for cp in L_cp(k + 1):
                cp.wait()
            compute_f(k + 1)
            for cp in G_cp(k + 1):
                cp.start()

        @pl.when(k + 2 < NCHUNKS)
        def _():
            for cp in L_cp(k + 2):
                cp.start()

        @pl.when(k >= 2)
        def _():
            S_cp(k - 2).wait()

        def erow(e, carry2):
            nbufs[b, e, pl.ds(0, L)] = 1.0 / (
                durows[b, e, pl.ds(0, L)] * dirows[b, e, pl.ds(0, L)])
            return carry2

        lax.fori_loop(0, CHUNK, erow, 0)
        S_cp(k).start()
        return carry

    lax.fori_loop(0, NCHUNKS, body, 0)
    S_cp(NCHUNKS - 2).wait()
    S_cp(NCHUNKS - 1).wait()


@functools.partial(
    pl.kernel,
    out_type=jax.ShapeDtypeStruct((NCORES, NU, D), _f32),
    mesh=_MESH,
    scratch_types=[
        pltpu.VMEM_SHARED((NU, D), _f32),
        pltpu.VMEM((4, CHUNK), _i32),
        pltpu.VMEM((4, CHUNK), _i32),
        pltpu.VMEM((2, CHUNK, L), _f32),
        pltpu.VMEM((2, CHUNK, D), _f32),
        pltpu.SemaphoreType.DMA((4,)),
        pltpu.SemaphoreType.DMA((2,)),
        pltpu.SemaphoreType.DMA((2,)),
    ],
)
def _dir_kernel(feats, gidx, sidx, nrm, zc, part, acc, gbufs, sbufs,
                nbufs, rows, semL, semG, semS):
    c = lax.axis_index("c")
    s = lax.axis_index("s")
    zb = jnp.minimum(s * FZ, NU - FZ)
    pltpu.sync_copy(zc.at[pl.ds(zb, FZ)], acc.at[pl.ds(zb, FZ)])
    plsc.subcore_barrier()

    tile_base = (c * NSUB + s) * E_PER_TILE

    def L_cp(k):
        sl = lax.rem(k, 4)
        base = tile_base + k * CHUNK
        return (
            pltpu.make_async_copy(gidx.at[pl.ds(base, CHUNK)], gbufs.at[sl],
                                  semL.at[sl]),
            pltpu.make_async_copy(sidx.at[pl.ds(base, CHUNK)], sbufs.at[sl],
                                  semL.at[sl]),
        )

    def G_cp(k):
        sl = lax.rem(k, 4)
        r = lax.rem(k, 2)
        base = tile_base + k * CHUNK
        return (
            pltpu.make_async_copy(feats.at[gbufs.at[sl]], rows.at[r],
                                  semG.at[r]),
            pltpu.make_async_copy(nrm.at[pl.ds(base, CHUNK)], nbufs.at[r],
                                  semG.at[r]),
        )

    def S_cp(k):
        sl = lax.rem(k, 4)
        r = lax.rem(k, 2)
        return pltpu.make_async_copy(rows.at[r], acc.at[sbufs.at[sl]],
                                     semS.at[r])

    def issue_S(k):
        sl = lax.rem(k, 4)
        r = lax.rem(k, 2)
        pltpu.async_copy(rows.at[r], acc.at[sbufs.at[sl]], semS.at[r],
                         add=True)

    for cp in L_cp(0):
        cp.start()
    for cp in L_cp(0):
        cp.wait()
    for cp in G_cp(0):
        cp.start()
    for cp in L_cp(1):
        cp.start()

    def body(k, carry):
        r = lax.rem(k, 2)
        for cp in G_cp(k):
            cp.wait()

        @pl.when(k + 1 < NCHUNKS)
        def _():
            @pl.when(k >= 1)
            def _():
                S_cp(k - 1).wait()
            for cp in L_cp(k + 1):
                cp.wait()
            for cp in G_cp(k + 1):
                cp.start()

        @pl.when(k + 2 < NCHUNKS)
        def _():
            for cp in L_cp(k + 2):
                cp.start()

        def escale(e, carry2):
            nsp = nbufs[r, e, :]
            for j in range(D // L):
                rows[r, e, pl.ds(j * L, L)] = rows[r, e, pl.ds(j * L, L)] * nsp
            return carry2

        lax.fori_loop(0, CHUNK, escale, 0)
        issue_S(k)
        return carry

    lax.fori_loop(0, NCHUNKS, body, 0)
    S_cp(NCHUNKS - 2).wait()
    S_cp(NCHUNKS - 1).wait()
    plsc.subcore_barrier()
    pltpu.sync_copy(acc.at[pl.ds(zb, FZ)], part.at[c, pl.ds(zb, FZ)])


@functools.partial(
    pl.kernel,
    out_type=(
        jax.ShapeDtypeStruct((ED, 2 * D), _f32),
        jax.ShapeDtypeStruct((ED, D), _f32),
    ),
    mesh=_MESH,
    scratch_types=[
        pltpu.VMEM((4, DCHUNK), _i32),
        pltpu.VMEM((4, DCHUNK), _i32),
        pltpu.VMEM((2, DCHUNK, 2 * D), _f32),
        pltpu.VMEM((2, DCHUNK, D), _f32),
        pltpu.SemaphoreType.DMA((4,)),
        pltpu.SemaphoreType.DMA((2,)),
        pltpu.SemaphoreType.DMA((2,)),
    ],
)
def _dec_kernel(zt, it_, d0, d1, zg, ig, dubufs, dibufs, zrows, irows, semL,
                semG, semS):
    c = lax.axis_index("c")
    s = lax.axis_index("s")
    wid = c * NSUB + s
    base0 = jnp.minimum(wid * DTILE, ED - DTILE)

    def L_cp(k):
        sl = lax.rem(k, 4)
        cb = base0 + k * DCHUNK
        return (
            pltpu.make_async_copy(d0.at[pl.ds(cb, DCHUNK)], dubufs.at[sl],
                                  semL.at[sl]),
            pltpu.make_async_copy(d1.at[pl.ds(cb, DCHUNK)], dibufs.at[sl],
                                  semL.at[sl]),
        )

    def G_cp(k):
        sl = lax.rem(k, 4)
        b = lax.rem(k, 2)
        return (
            pltpu.make_async_copy(zt.at[dubufs.at[sl]], zrows.at[b],
                                  semG.at[b]),
            pltpu.make_async_copy(it_.at[dibufs.at[sl]], irows.at[b],
                                  semG.at[b]),
        )

    def S_cp(k):
        b = lax.rem(k, 2)
        cb = base0 + k * DCHUNK
        return (
            pltpu.make_async_copy(zrows.at[b], zg.at[pl.ds(cb, DCHUNK)],
                                  semS.at[b]),
            pltpu.make_async_copy(irows.at[b], ig.at[pl.ds(cb, DCHUNK)],
                                  semS.at[b]),
        )

    for cp in L_cp(0):
        cp.start()
    for cp in L_cp(0):
        cp.wait()
    for cp in G_cp(0):
        cp.start()
    for cp in L_cp(1):
        cp.start()

    def body(k, carry):
        for cp in G_cp(k):
            cp.wait()

        @pl.when(k + 1 < DNCH)
        def _():
            @pl.when(k >= 1)
            def _():
                for cp in S_cp(k - 1):
                    cp.wait()
            for cp in L_cp(k + 1):
                cp.wait()
            for cp in G_cp(k + 1):
                cp.start()

        @pl.when(k + 2 < DNCH)
        def _():
            for cp in L_cp(k + 2):
                cp.start()

        for cp in S_cp(k):
            cp.start()
        return carry

    lax.fori_loop(0, DNCH, body, 0)
    for cp in S_cp(DNCH - 2):
        cp.wait()
    for cp in S_cp(DNCH - 1):
        cp.wait()


def _sumui_body(pu_ref, pi_ref, u_ref, i_ref):
    u_ref[...] = pu_ref[0] + pu_ref[1]
    i_ref[...] = pi_ref[0] + pi_ref[1]


def _sumui(pu, pi):
    blk = NU // 5
    spec_in = pl.BlockSpec((NCORES, blk, D), lambda i: (0, i, 0))
    spec_out = pl.BlockSpec((blk, D), lambda i: (i, 0))
    return pl.pallas_call(
        _sumui_body,
        grid=(5,),
        in_specs=[spec_in, spec_in],
        out_specs=(spec_out, spec_out),
        out_shape=(
            jax.ShapeDtypeStruct((NU, D), _f32),
            jax.ShapeDtypeStruct((NI, D), _f32),
        ),
    )(pu, pi)


_XBLK = 1000


def _expand_body(u_ref, i_ref, ou_ref, oi_ref):
    r = pl.program_id(0) // (NU // _XBLK)
    lanes = lax.broadcasted_iota(_i32, (_XBLK, D), 1)
    mask = (lanes == r).astype(_f32)
    cu = jnp.sum(u_ref[...] * mask, axis=1)
    ci = jnp.sum(i_ref[...] * mask, axis=1)
    ou_ref[...] = jnp.broadcast_to(cu[:, None], (_XBLK, D))
    oi_ref[...] = jnp.broadcast_to(ci[:, None], (_XBLK, D))


def _expand(degu, degi):
    nb = NU // _XBLK
    spec_in = pl.BlockSpec((_XBLK, D), lambda b: (b % nb, 0))
    spec_out = pl.BlockSpec((_XBLK, D), lambda b: (b, 0))
    return pl.pallas_call(
        _expand_body,
        grid=(N_CLASSES * nb,),
        in_specs=[spec_in, spec_in],
        out_specs=(spec_out, spec_out),
        out_shape=(
            jax.ShapeDtypeStruct((N_CLASSES * NU, D), _f32),
            jax.ShapeDtypeStruct((N_CLASSES * NI, D), _f32),
        ),
    )(degu, degi)


def _zmm_body(pu_ref, pi_ref, pc_ref, z_ref, i_ref):
    u2 = pu_ref[0] + pu_ref[1]
    z_ref[...] = jnp.dot(u2, pc_ref[...], preferred_element_type=_f32)
    i_ref[...] = pi_ref[0] + pi_ref[1]


def _zmm(pu, pi, pcat):
    blk = NU // 5
    spec_in = pl.BlockSpec((NCORES, blk, D), lambda i: (0, i, 0))
    return pl.pallas_call(
        _zmm_body,
        grid=(5,),
        in_specs=[
            spec_in,
            spec_in,
            pl.BlockSpec((D, 2 * D), lambda i: (0, 0)),
        ],
        out_specs=(
            pl.BlockSpec((blk, 2 * D), lambda i: (i, 0)),
            pl.BlockSpec((blk, D), lambda i: (i, 0)),
        ),
        out_shape=(
            jax.ShapeDtypeStruct((NU, 2 * D), _f32),
            jax.ShapeDtypeStruct((NI, D), _f32),
        ),
    )(pu, pi, pcat)


def _dot_body(zg_ref, ig_ref, c_ref, o_ref):
    ig = ig_ref[...]
    s0 = jnp.sum(zg_ref[:, :D] * ig, axis=1)
    s1 = jnp.sum(zg_ref[:, D:] * ig, axis=1)
    c0 = c_ref[:, 0]
    c1 = c_ref[:, 1]
    o_ref[...] = s0[:, None] * c0[None, :] + s1[:, None] * c1[None, :]


def _dots(zg, ig, coef):
    blk = ED // 25
    return pl.pallas_call(
        _dot_body,
        grid=(25,),
        in_specs=[
            pl.BlockSpec((blk, 2 * D), lambda i: (i, 0)),
            pl.BlockSpec((blk, D), lambda i: (i, 0)),
            pl.BlockSpec((N_CLASSES, 2), lambda i: (0, 0)),
        ],
        out_specs=pl.BlockSpec((blk, N_CLASSES), lambda i: (i, 0)),
        out_shape=jax.ShapeDtypeStruct((ED, N_CLASSES), _f32),
    )(zg, ig, coef)


def kernel(ufeats, ifeats, P_basis, coef, enc_edge_index, enc_edge_types,
           dec_edge_index):
    eu = enc_edge_index[0].astype(_i32)
    ei = enc_edge_index[1].astype(_i32)
    et = enc_edge_types.astype(_i32)
    d0 = dec_edge_index[0].astype(_i32)
    d1 = dec_edge_index[1].astype(_i32)

    zfeat = jnp.zeros((NU, D), _f32)

    dpu, dpi = _deg_kernel(eu, ei, et, zfeat)
    degu, degi = _sumui(dpu, dpi)
    degux, degix = _expand(degu, degi)
    nrm = _norm_kernel(eu, ei, et, degux, degix)

    pu = _dir_kernel(ifeats, ei, eu, nrm, zfeat)
    pi = _dir_kernel(ufeats, eu, ei, nrm, zfeat)
    u1, i1 = _sumui(pu, pi)
    pu2 = _dir_kernel(i1, ei, eu, nrm, zfeat)
    pi2 = _dir_kernel(u1, eu, ei, nrm, zfeat)

    pcat = jnp.concatenate([P_basis[0], P_basis[1]], axis=1)
    zcat, i2 = _zmm(pu2, pi2, pcat)

    zg, ig = _dec_kernel(zcat, i2, d0, d1)
    return _dots(zg, ig, coef)


# speedup vs baseline: 1.7432x; 1.7432x over previous
import functools

import jax
import jax.numpy as jnp
from jax import lax
from jax.experimental import pallas as pl
from jax.experimental.pallas import tpu as pltpu
from jax.experimental.pallas import tpu_sc as plsc

N_CLASSES = 5
D = 128
NU = 5000
NI = 5000
E = 320000
ED = 100000

NCORES = 2
NSUB = 16
L = 16

E_PER_TILE = E // (NCORES * NSUB)
CHUNK = 80
NCHUNKS = E_PER_TILE // CHUNK

FZ = 320

DTILE = 3136
DCHUNK = 112
DNCH = DTILE // DCHUNK

_MESH = plsc.VectorSubcoreMesh(
    core_axis_name="c", subcore_axis_name="s",
    num_cores=NCORES, num_subcores=NSUB)

_f32 = jnp.float32
_i32 = jnp.int32


def _dg(v, idx):
    return lax.gather(
        v, idx.reshape(L, 1),
        lax.GatherDimensionNumbers(
            offset_dims=(), collapsed_slice_dims=(0,), start_index_map=(0,)),
        slice_sizes=(1,),
        mode=lax.GatherScatterMode.PROMISE_IN_BOUNDS)


@functools.partial(
    pl.kernel,
    out_type=(
        jax.ShapeDtypeStruct((NCORES, NU, D), _f32),
        jax.ShapeDtypeStruct((NCORES, NI, D), _f32),
    ),
    mesh=_MESH,
    scratch_types=[
        pltpu.VMEM_SHARED((NU, D), _f32),
        pltpu.VMEM_SHARED((NI, D), _f32),
        pltpu.VMEM((4, CHUNK), _i32),
        pltpu.VMEM((4, CHUNK), _i32),
        pltpu.VMEM((4, CHUNK), _i32),
        pltpu.VMEM((2, CHUNK, D), _f32),
        pltpu.SemaphoreType.DMA((4,)),
        pltpu.SemaphoreType.DMA((2,)),
    ],
)
def _deg_kernel(eu, ei, et, zc, dpu, dpi, tabu, tabi, ubufs, ibufs, tbufs,
                msgs, semL, semS):
    c = lax.axis_index("c")
    s = lax.axis_index("s")
    zb = jnp.minimum(s * FZ, NU - FZ)
    pltpu.sync_copy(zc.at[pl.ds(zb, FZ)], tabu.at[pl.ds(zb, FZ)])
    pltpu.sync_copy(zc.at[pl.ds(zb, FZ)], tabi.at[pl.ds(zb, FZ)])
    pltpu.sync_copy(zc.at[pl.ds(0, CHUNK)], msgs.at[0])
    pltpu.sync_copy(zc.at[pl.ds(0, CHUNK)], msgs.at[1])
    plsc.subcore_barrier()

    tile_base = (c * NSUB + s) * E_PER_TILE
    iota = lax.iota(_i32, L)

    def L_cp(k):
        sl = lax.rem(k, 4)
        base = tile_base + k * CHUNK
        return (
            pltpu.make_async_copy(et.at[pl.ds(base, CHUNK)], tbufs.at[sl],
                                  semL.at[sl]),
            pltpu.make_async_copy(eu.at[pl.ds(base, CHUNK)], ubufs.at[sl],
                                  semL.at[sl]),
            pltpu.make_async_copy(ei.at[pl.ds(base, CHUNK)], ibufs.at[sl],
                                  semL.at[sl]),
        )

    def issue_L(k):
        for cp in L_cp(k):
            cp.start()

    def wait_L(k):
        for cp in L_cp(k):
            cp.wait()

    def issue_S(k):
        sl = lax.rem(k, 4)
        b = lax.rem(k, 2)
        pltpu.async_copy(msgs.at[b], tabu.at[ubufs.at[sl]], semS.at[b],
                         add=True)
        pltpu.async_copy(msgs.at[b], tabi.at[ibufs.at[sl]], semS.at[b],
                         add=True)

    def wait_S(k):
        sl = lax.rem(k, 4)
        b = lax.rem(k, 2)
        pltpu.make_async_copy(msgs.at[b], tabu.at[ubufs.at[sl]],
                              semS.at[b]).wait()
        pltpu.make_async_copy(msgs.at[b], tabi.at[ibufs.at[sl]],
                              semS.at[b]).wait()

    issue_L(0)
    issue_L(1)

    def body(k, carry):
        sl = lax.rem(k, 4)
        b = lax.rem(k, 2)
        wait_L(k)

        @pl.when(k >= 2)
        def _():
            wait_S(k - 2)

        for g in range(CHUNK // L):
            et16 = tbufs[sl, pl.ds(g * L, L)]
            for l in range(L):
                et_sp = _dg(et16, jnp.full((L,), l, _i32))
                msgs[b, g * L + l, pl.ds(0, L)] = jnp.where(
                    iota == et_sp, 1.0, 0.0)
        issue_S(k)

        @pl.when(k + 2 < NCHUNKS)
        def _():
            issue_L(k + 2)

        return carry

    lax.fori_loop(0, NCHUNKS, body, 0)
    wait_S(NCHUNKS - 2)
    wait_S(NCHUNKS - 1)
    plsc.subcore_barrier()
    pltpu.sync_copy(tabu.at[pl.ds(zb, FZ)], dpu.at[c, pl.ds(zb, FZ)])
    pltpu.sync_copy(tabi.at[pl.ds(zb, FZ)], dpi.at[c, pl.ds(zb, FZ)])


@functools.partial(
    pl.kernel,
    out_type=jax.ShapeDtypeStruct((E, L), _f32),
    mesh=_MESH,
    scratch_types=[
        pltpu.VMEM((4, CHUNK), _i32),
        pltpu.VMEM((4, CHUNK), _i32),
        pltpu.VMEM((4, CHUNK), _i32),
        pltpu.VMEM((4, CHUNK), _i32),
        pltpu.VMEM((2, CHUNK, D), _f32),
        pltpu.VMEM((2, CHUNK, D), _f32),
        pltpu.VMEM((2, CHUNK, L), _f32),
        pltpu.SemaphoreType.DMA((4,)),
        pltpu.SemaphoreType.DMA((2,)),
        pltpu.SemaphoreType.DMA((2,)),
    ],
)
def _norm_kernel(eu, ei, et, degux, degix, nrm, ebufs, tbufs, fubufs, fibufs,
                 durows, dirows, nbufs, semL, semG, semS):
    c = lax.axis_index("c")
    s = lax.axis_index("s")
    tile_base = (c * NSUB + s) * E_PER_TILE

    def L_cp(k):
        sl = lax.rem(k, 4)
        base = tile_base + k * CHUNK
        return (
            pltpu.make_async_copy(et.at[pl.ds(base, CHUNK)], tbufs.at[sl],
                                  semL.at[sl]),
            pltpu.make_async_copy(eu.at[pl.ds(base, CHUNK)], ebufs.at[sl],
                                  semL.at[sl]),
        )

    def Li_cp(k):
        sl = lax.rem(k, 4)
        base = tile_base + k * CHUNK
        return pltpu.make_async_copy(ei.at[pl.ds(base, CHUNK)],
                                     ebufs.at[sl], semL.at[sl])

    def compute_f(k):
        sl = lax.rem(k, 4)
        for g in range(CHUNK // L):
            t16 = tbufs[sl, pl.ds(g * L, L)]
            u16 = ebufs[sl, pl.ds(g * L, L)]
            fubufs[sl, pl.ds(g * L, L)] = t16 * NU + u16
        Li_cp(k).start()
        Li_cp(k).wait()
        for g in range(CHUNK // L):
            t16 = tbufs[sl, pl.ds(g * L, L)]
            i16 = ebufs[sl, pl.ds(g * L, L)]
            fibufs[sl, pl.ds(g * L, L)] = t16 * NU + i16

    def G_cp(k):
        sl = lax.rem(k, 4)
        b = lax.rem(k, 2)
        return (
            pltpu.make_async_copy(degux.at[fubufs.at[sl]], durows.at[b],
                                  semG.at[b]),
            pltpu.make_async_copy(degix.at[fibufs.at[sl]], dirows.at[b],
                                  semG.at[b]),
        )

    def S_cp(k):
        b = lax.rem(k, 2)
        base = tile_base + k * CHUNK
        return pltpu.make_async_copy(nbufs.at[b],
                                     nrm.at[pl.ds(base, CHUNK)], semS.at[b])

    for cp in L_cp(0):
        cp.start()
    for cp in L_cp(0):
        cp.wait()
    compute_f(0)
    for cp in G_cp(0):
        cp.start()
    for cp in L_cp(1):
        cp.start()

    def body(k, carry):
        b = lax.rem(k, 2)
        for cp in G_cp(k):
            cp.wait()

        @pl.when(k + 1 < NCHUNKS)
        def _():
            for cp in L_cp(k + 1):
                cp.wait()
            compute_f(k + 1)
            for cp in G_cp(k + 1):
                cp.start()

        @pl.when(k + 2 < NCHUNKS)
        def _():
            for cp in L_cp(k + 2):
                cp.start()

        @pl.when(k >= 2)
        def _():
            S_cp(k - 2).wait()

        def erow(e, carry2):
            nbufs[b, e, pl.ds(0, L)] = 1.0 / (
                durows[b, e, pl.ds(0, L)] * dirows[b, e, pl.ds(0, L)])
            return carry2

        lax.fori_loop(0, CHUNK, erow, 0)
        S_cp(k).start()
        return carry

    lax.fori_loop(0, NCHUNKS, body, 0)
    S_cp(NCHUNKS - 2).wait()
    S_cp(NCHUNKS - 1).wait()


@functools.partial(
    pl.kernel,
    out_type=jax.ShapeDtypeStruct((NCORES, NU, D), _f32),
    mesh=_MESH,
    scratch_types=[
        pltpu.VMEM_SHARED((NU, D), _f32),
        pltpu.VMEM((4, CHUNK), _i32),
        pltpu.VMEM((4, CHUNK), _i32),
        pltpu.VMEM((2, CHUNK, L), _f32),
        pltpu.VMEM((2, CHUNK, D), _f32),
        pltpu.SemaphoreType.DMA((4,)),
        pltpu.SemaphoreType.DMA((2,)),
        pltpu.SemaphoreType.DMA((2,)),
    ],
)
def _dir_kernel(feats, gidx, sidx, nrm, zc, part, acc, gbufs, sbufs,
                nbufs, rows, semL, semG, semS):
    c = lax.axis_index("c")
    s = lax.axis_index("s")
    zb = jnp.minimum(s * FZ, NU - FZ)
    pltpu.sync_copy(zc.at[pl.ds(zb, FZ)], acc.at[pl.ds(zb, FZ)])
    plsc.subcore_barrier()

    tile_base = (c * NSUB + s) * E_PER_TILE

    def L_cp(k):
        sl = lax.rem(k, 4)
        base = tile_base + k * CHUNK
        return (
            pltpu.make_async_copy(gidx.at[pl.ds(base, CHUNK)], gbufs.at[sl],
                                  semL.at[sl]),
            pltpu.make_async_copy(sidx.at[pl.ds(base, CHUNK)], sbufs.at[sl],
                                  semL.at[sl]),
        )

    def G_cp(k):
        sl = lax.rem(k, 4)
        r = lax.rem(k, 2)
        base = tile_base + k * CHUNK
        return (
            pltpu.make_async_copy(feats.at[gbufs.at[sl]], rows.at[r],
                                  semG.at[r]),
            pltpu.make_async_copy(nrm.at[pl.ds(base, CHUNK)], nbufs.at[r],
                                  semG.at[r]),
        )

    def S_cp(k):
        sl = lax.rem(k, 4)
        r = lax.rem(k, 2)
        return pltpu.make_async_copy(rows.at[r], acc.at[sbufs.at[sl]],
                                     semS.at[r])

    def issue_S(k):
        sl = lax.rem(k, 4)
        r = lax.rem(k, 2)
        pltpu.async_copy(rows.at[r], acc.at[sbufs.at[sl]], semS.at[r],
                         add=True)

    for cp in L_cp(0):
        cp.start()
    for cp in L_cp(0):
        cp.wait()
    for cp in G_cp(0):
        cp.start()
    for cp in L_cp(1):
        cp.start()

    def body(k, carry):
        r = lax.rem(k, 2)
        for cp in G_cp(k):
            cp.wait()

        @pl.when(k + 1 < NCHUNKS)
        def _():
            @pl.when(k >= 1)
            def _():
                S_cp(k - 1).wait()
            for cp in L_cp(k + 1):
                cp.wait()
            for cp in G_cp(k + 1):
                cp.start()

        @pl.when(k + 2 < NCHUNKS)
        def _():
            for cp in L_cp(k + 2):
                cp.start()

        for e in range(CHUNK):
            nsp = nbufs[r, e, :]
            for j in range(D // L):
                rows[r, e, pl.ds(j * L, L)] = rows[r, e, pl.ds(j * L, L)] * nsp
        issue_S(k)
        return carry

    lax.fori_loop(0, NCHUNKS, body, 0)
    S_cp(NCHUNKS - 2).wait()
    S_cp(NCHUNKS - 1).wait()
    plsc.subcore_barrier()
    pltpu.sync_copy(acc.at[pl.ds(zb, FZ)], part.at[c, pl.ds(zb, FZ)])


@functools.partial(
    pl.kernel,
    out_type=(
        jax.ShapeDtypeStruct((ED, 2 * D), _f32),
        jax.ShapeDtypeStruct((ED, D), _f32),
    ),
    mesh=_MESH,
    scratch_types=[
        pltpu.VMEM((4, DCHUNK), _i32),
        pltpu.VMEM((4, DCHUNK), _i32),
        pltpu.VMEM((2, DCHUNK, 2 * D), _f32),
        pltpu.VMEM((2, DCHUNK, D), _f32),
        pltpu.SemaphoreType.DMA((4,)),
        pltpu.SemaphoreType.DMA((2,)),
        pltpu.SemaphoreType.DMA((2,)),
    ],
)
def _dec_kernel(zt, it_, d0, d1, zg, ig, dubufs, dibufs, zrows, irows, semL,
                semG, semS):
    c = lax.axis_index("c")
    s = lax.axis_index("s")
    wid = c * NSUB + s
    base0 = jnp.minimum(wid * DTILE, ED - DTILE)

    def L_cp(k):
        sl = lax.rem(k, 4)
        cb = base0 + k * DCHUNK
        return (
            pltpu.make_async_copy(d0.at[pl.ds(cb, DCHUNK)], dubufs.at[sl],
                                  semL.at[sl]),
            pltpu.make_async_copy(d1.at[pl.ds(cb, DCHUNK)], dibufs.at[sl],
                                  semL.at[sl]),
        )

    def G_cp(k):
        sl = lax.rem(k, 4)
        b = lax.rem(k, 2)
        return (
            pltpu.make_async_copy(zt.at[dubufs.at[sl]], zrows.at[b],
                                  semG.at[b]),
            pltpu.make_async_copy(it_.at[dibufs.at[sl]], irows.at[b],
                                  semG.at[b]),
        )

    def S_cp(k):
        b = lax.rem(k, 2)
        cb = base0 + k * DCHUNK
        return (
            pltpu.make_async_copy(zrows.at[b], zg.at[pl.ds(cb, DCHUNK)],
                                  semS.at[b]),
            pltpu.make_async_copy(irows.at[b], ig.at[pl.ds(cb, DCHUNK)],
                                  semS.at[b]),
        )

    for cp in L_cp(0):
        cp.start()
    for cp in L_cp(0):
        cp.wait()
    for cp in G_cp(0):
        cp.start()
    for cp in L_cp(1):
        cp.start()

    def body(k, carry):
        for cp in G_cp(k):
            cp.wait()

        @pl.when(k + 1 < DNCH)
        def _():
            @pl.when(k >= 1)
            def _():
                for cp in S_cp(k - 1):
                    cp.wait()
            for cp in L_cp(k + 1):
                cp.wait()
            for cp in G_cp(k + 1):
                cp.start()

        @pl.when(k + 2 < DNCH)
        def _():
            for cp in L_cp(k + 2):
                cp.start()

        for cp in S_cp(k):
            cp.start()
        return carry

    lax.fori_loop(0, DNCH, body, 0)
    for cp in S_cp(DNCH - 2):
        cp.wait()
    for cp in S_cp(DNCH - 1):
        cp.wait()


def _sumui_body(pu_ref, pi_ref, u_ref, i_ref):
    u_ref[...] = pu_ref[0] + pu_ref[1]
    i_ref[...] = pi_ref[0] + pi_ref[1]


def _sumui(pu, pi):
    blk = NU // 5
    spec_in = pl.BlockSpec((NCORES, blk, D), lambda i: (0, i, 0))
    spec_out = pl.BlockSpec((blk, D), lambda i: (i, 0))
    return pl.pallas_call(
        _sumui_body,
        grid=(5,),
        in_specs=[spec_in, spec_in],
        out_specs=(spec_out, spec_out),
        out_shape=(
            jax.ShapeDtypeStruct((NU, D), _f32),
            jax.ShapeDtypeStruct((NI, D), _f32),
        ),
    )(pu, pi)


_XBLK = 1000


def _expand_body(u_ref, i_ref, ou_ref, oi_ref):
    r = pl.program_id(0) // (NU // _XBLK)
    lanes = lax.broadcasted_iota(_i32, (_XBLK, D), 1)
    mask = (lanes == r).astype(_f32)
    cu = jnp.sum(u_ref[...] * mask, axis=1)
    ci = jnp.sum(i_ref[...] * mask, axis=1)
    ou_ref[...] = jnp.broadcast_to(cu[:, None], (_XBLK, D))
    oi_ref[...] = jnp.broadcast_to(ci[:, None], (_XBLK, D))


def _expand(degu, degi):
    nb = NU // _XBLK
    spec_in = pl.BlockSpec((_XBLK, D), lambda b: (b % nb, 0))
    spec_out = pl.BlockSpec((_XBLK, D), lambda b: (b, 0))
    return pl.pallas_call(
        _expand_body,
        grid=(N_CLASSES * nb,),
        in_specs=[spec_in, spec_in],
        out_specs=(spec_out, spec_out),
        out_shape=(
            jax.ShapeDtypeStruct((N_CLASSES * NU, D), _f32),
            jax.ShapeDtypeStruct((N_CLASSES * NI, D), _f32),
        ),
    )(degu, degi)


def _zmm_body(pu_ref, pi_ref, pc_ref, z_ref, i_ref):
    u2 = pu_ref[0] + pu_ref[1]
    z_ref[...] = jnp.dot(u2, pc_ref[...], preferred_element_type=_f32)
    i_ref[...] = pi_ref[0] + pi_ref[1]


def _zmm(pu, pi, pcat):
    blk = NU // 5
    spec_in = pl.BlockSpec((NCORES, blk, D), lambda i: (0, i, 0))
    return pl.pallas_call(
        _zmm_body,
        grid=(5,),
        in_specs=[
            spec_in,
            spec_in,
            pl.BlockSpec((D, 2 * D), lambda i: (0, 0)),
        ],
        out_specs=(
            pl.BlockSpec((blk, 2 * D), lambda i: (i, 0)),
            pl.BlockSpec((blk, D), lambda i: (i, 0)),
        ),
        out_shape=(
            jax.ShapeDtypeStruct((NU, 2 * D), _f32),
            jax.ShapeDtypeStruct((NI, D), _f32),
        ),
    )(pu, pi, pcat)


def _dot_body(zg_ref, ig_ref, c_ref, o_ref):
    ig = ig_ref[...]
    s0 = jnp.sum(zg_ref[:, :D] * ig, axis=1)
    s1 = jnp.sum(zg_ref[:, D:] * ig, axis=1)
    c0 = c_ref[:, 0]
    c1 = c_ref[:, 1]
    o_ref[...] = s0[:, None] * c0[None, :] + s1[:, None] * c1[None, :]


def _dots(zg, ig, coef):
    blk = ED // 25
    return pl.pallas_call(
        _dot_body,
        grid=(25,),
        in_specs=[
            pl.BlockSpec((blk, 2 * D), lambda i: (i, 0)),
            pl.BlockSpec((blk, D), lambda i: (i, 0)),
            pl.BlockSpec((N_CLASSES, 2), lambda i: (0, 0)),
        ],
        out_specs=pl.BlockSpec((blk, N_CLASSES), lambda i: (i, 0)),
        out_shape=jax.ShapeDtypeStruct((ED, N_CLASSES), _f32),
    )(zg, ig, coef)


def kernel(ufeats, ifeats, P_basis, coef, enc_edge_index, enc_edge_types,
           dec_edge_index):
    eu = enc_edge_index[0].astype(_i32)
    ei = enc_edge_index[1].astype(_i32)
    et = enc_edge_types.astype(_i32)
    d0 = dec_edge_index[0].astype(_i32)
    d1 = dec_edge_index[1].astype(_i32)

    zfeat = jnp.zeros((NU, D), _f32)

    dpu, dpi = _deg_kernel(eu, ei, et, zfeat)
    degu, degi = _sumui(dpu, dpi)
    degux, degix = _expand(degu, degi)
    nrm = _norm_kernel(eu, ei, et, degux, degix)

    pu = _dir_kernel(ifeats, ei, eu, nrm, zfeat)
    pi = _dir_kernel(ufeats, eu, ei, nrm, zfeat)
    u1, i1 = _sumui(pu, pi)
    pu2 = _dir_kernel(i1, ei, eu, nrm, zfeat)
    pi2 = _dir_kernel(u1, eu, ei, nrm, zfeat)

    pcat = jnp.concatenate([P_basis[0], P_basis[1]], axis=1)
    zcat, i2 = _zmm(pu2, pi2, pcat)

    zg, ig = _dec_kernel(zcat, i2, d0, d1)
    return _dots(zg, ig, coef)
